# Initial kernel scaffold; baseline (speedup 1.0000x reference)
import jax
import jax.numpy as jnp
from jax.experimental import pallas as pl


def kernel(img, W_spix, b_spix, W_patch, b_patch, w_attn, W_out, b_out):
    raise NotImplementedError("write your pallas kernel here")


# speedup vs baseline: 57.1990x; 57.1990x over previous
import dataclasses
import functools

import jax
import jax.numpy as jnp
import numpy as np
from jax import lax
from jax.experimental import pallas as pl
from jax.experimental.pallas import tpu as pltpu
from jax.experimental.pallas import tpu_sc as plsc

F32 = jnp.float32
I32 = jnp.int32


def _sc_compiler_params():
    cp = pltpu.CompilerParams()
    if "needs_layout_passes" in pltpu.CompilerParams.__dataclass_fields__:
        cp = dataclasses.replace(cp, needs_layout_passes=False)
    return cp


def _k1_body(x_ref, w_ref, b_ref, wa_ref, wp_ref, lab_ref, wbar_ref):
    x = x_ref[0]
    logits = jnp.dot(x, w_ref[...], preferred_element_type=F32) + b_ref[...]
    lab_ref[0, 0, :] = jnp.argmax(logits, axis=1).astype(I32)

    @pl.when(jnp.logical_and(pl.program_id(0) == 0, pl.program_id(1) == 0))
    def _():
        wbar_ref[...] = jnp.dot(wa_ref[...], wp_ref[...],
                                preferred_element_type=F32)


def _run_k1(X, W27, b64, wa, Wp2, B, HW, M):
    nb = HW // M
    return pl.pallas_call(
        _k1_body,
        grid=(B, nb),
        in_specs=[
            pl.BlockSpec((1, M, 32), lambda b, i: (b, i, 0)),
            pl.BlockSpec((32, 64), lambda b, i: (0, 0)),
            pl.BlockSpec((1, 64), lambda b, i: (0, 0)),
            pl.BlockSpec((1, 96), lambda b, i: (0, 0)),
            pl.BlockSpec((96, 768), lambda b, i: (0, 0)),
        ],
        out_specs=[
            pl.BlockSpec((1, 1, M), lambda b, i: (b * nb + i, 0, 0)),
            pl.BlockSpec((1, 768), lambda b, i: (0, 0)),
        ],
        out_shape=[
            jax.ShapeDtypeStruct((B * nb, 1, M), I32),
            jax.ShapeDtypeStruct((1, 768), F32),
        ],
    )(X, W27, b64, wa, Wp2)


def _run_k2(labels, pid, off, img_flat, wbar, zeros256, B, HW, TPI, NPIX):
    mesh = plsc.VectorSubcoreMesh(core_axis_name="c", subcore_axis_name="s")

    @functools.partial(
        pl.kernel,
        mesh=mesh,
        compiler_params=_sc_compiler_params(),
        out_type=[
            jax.ShapeDtypeStruct((B, TPI, 64, 256), F32),
            jax.ShapeDtypeStruct((B * TPI * 64,), F32),
        ],
        scratch_types=[
            pltpu.VMEM((NPIX,), I32),
            pltpu.VMEM((NPIX,), I32),
            pltpu.VMEM((NPIX,), I32),
            pltpu.VMEM((NPIX,), F32),
            pltpu.VMEM((NPIX,), F32),
            pltpu.VMEM((NPIX,), F32),
            pltpu.VMEM((768,), F32),
            pltpu.VMEM((64, 256), F32),
            pltpu.VMEM((64,), F32),
        ],
    )
    def k2(lab_hbm, pid_hbm, off_hbm, img_hbm, wbar_hbm, z_hbm,
           s_out, c_out,
           lbl_v, pid_v, off_v, i0_v, i1_v, i2_v, wbar_v, sacc, cacc):
        wid = lax.axis_index("c") * 16 + lax.axis_index("s")
        b = wid // TPI
        part = wid % TPI
        base = part * NPIX
        pltpu.sync_copy(lab_hbm.at[pl.ds(b * HW + base, NPIX)], lbl_v)
        pltpu.sync_copy(pid_hbm.at[pl.ds(base, NPIX)], pid_v)
        pltpu.sync_copy(off_hbm.at[pl.ds(base, NPIX)], off_v)
        pltpu.sync_copy(img_hbm.at[pl.ds((b * 3 + 0) * HW + base, NPIX)], i0_v)
        pltpu.sync_copy(img_hbm.at[pl.ds((b * 3 + 1) * HW + base, NPIX)], i1_v)
        pltpu.sync_copy(img_hbm.at[pl.ds((b * 3 + 2) * HW + base, NPIX)], i2_v)
        pltpu.sync_copy(wbar_hbm, wbar_v)
        pltpu.sync_copy(z_hbm, sacc)

        @pl.loop(0, 64, step=16)
        def _(j):
            cacc[pl.ds(j, 16)] = jnp.zeros((16,), F32)

        ones = jnp.full((16,), 1.0, F32)

        @pl.loop(0, NPIX, step=16)
        def _(i):
            sl = pl.ds(i, 16)
            lbl = lbl_v[sl]
            pidv = pid_v[sl]
            offv = off_v[sl]
            w0 = plsc.load_gather(wbar_v, [offv])
            w1 = plsc.load_gather(wbar_v, [offv + 256])
            w2 = plsc.load_gather(wbar_v, [offv + 512])
            u = i0_v[sl] * w0 + i1_v[sl] * w1 + i2_v[sl] * w2
            plsc.addupdate_scatter(sacc, [lbl, pidv], u)
            plsc.addupdate_scatter(cacc, [lbl], ones)

        pltpu.sync_copy(sacc, s_out.at[b, part])
        pltpu.sync_copy(cacc, c_out.at[pl.ds(wid * 64, 64)])

    return k2(labels, pid, off, img_flat, wbar, zeros256)


def _k3_body(sp_ref, attn_ref):
    s = jnp.sum(sp_ref[0], axis=0)
    lane = lax.broadcasted_iota(I32, (64, 256), 1)
    valid = lane < 196
    neg = jnp.where(valid, s, -1e30)
    m = jnp.max(neg, axis=1, keepdims=True)
    e = jnp.where(valid, jnp.exp(s - m), 0.0)
    attn_ref[0] = e / jnp.sum(e, axis=1, keepdims=True)


def _run_k3(scores_part, B, TPI):
    return pl.pallas_call(
        _k3_body,
        grid=(B,),
        in_specs=[pl.BlockSpec((1, TPI, 64, 256), lambda b: (b, 0, 0, 0))],
        out_specs=pl.BlockSpec((1, 64, 256), lambda b: (b, 0, 0)),
        out_shape=jax.ShapeDtypeStruct((B, 64, 256), F32),
    )(scores_part)


def _run_k4(labels, pid, off, img_flat, attn, zeros768, B, HW, TPI, NPIX):
    mesh = plsc.VectorSubcoreMesh(core_axis_name="c", subcore_axis_name="s")

    @functools.partial(
        pl.kernel,
        mesh=mesh,
        compiler_params=_sc_compiler_params(),
        out_type=jax.ShapeDtypeStruct((B, TPI, 64, 768), F32),
        scratch_types=[
            pltpu.VMEM((NPIX,), I32),
            pltpu.VMEM((NPIX,), I32),
            pltpu.VMEM((NPIX,), I32),
            pltpu.VMEM((NPIX,), F32),
            pltpu.VMEM((NPIX,), F32),
            pltpu.VMEM((NPIX,), F32),
            pltpu.VMEM((64, 256), F32),
            pltpu.VMEM((64, 768), F32),
        ],
    )
    def k4(lab_hbm, pid_hbm, off_hbm, img_hbm, attn_hbm, z_hbm, zw_out,
           lbl_v, pid_v, off_v, i0_v, i1_v, i2_v, attn_v, zacc):
        wid = lax.axis_index("c") * 16 + lax.axis_index("s")
        b = wid // TPI
        part = wid % TPI
        base = part * NPIX
        pltpu.sync_copy(lab_hbm.at[pl.ds(b * HW + base, NPIX)], lbl_v)
        pltpu.sync_copy(pid_hbm.at[pl.ds(base, NPIX)], pid_v)
        pltpu.sync_copy(off_hbm.at[pl.ds(base, NPIX)], off_v)
        pltpu.sync_copy(img_hbm.at[pl.ds((b * 3 + 0) * HW + base, NPIX)], i0_v)
        pltpu.sync_copy(img_hbm.at[pl.ds((b * 3 + 1) * HW + base, NPIX)], i1_v)
        pltpu.sync_copy(img_hbm.at[pl.ds((b * 3 + 2) * HW + base, NPIX)], i2_v)
        pltpu.sync_copy(attn_hbm.at[b], attn_v)
        pltpu.sync_copy(z_hbm, zacc)

        @pl.loop(0, NPIX, step=16)
        def _(i):
            sl = pl.ds(i, 16)
            lbl = lbl_v[sl]
            pidv = pid_v[sl]
            offv = off_v[sl]
            a = plsc.load_gather(attn_v, [lbl, pidv])
            plsc.addupdate_scatter(zacc, [lbl, offv], i0_v[sl] * a)
            plsc.addupdate_scatter(zacc, [lbl, offv + 256], i1_v[sl] * a)
            plsc.addupdate_scatter(zacc, [lbl, offv + 512], i2_v[sl] * a)

        pltpu.sync_copy(zacc, zw_out.at[b, part])

    return k4(labels, pid, off, img_flat, attn, zeros768)


def _k5_body(zw_ref, cnt_ref, wq_ref, bp_ref, wo_ref, bo_ref, out_ref):
    Z = jnp.sum(zw_ref[0], axis=0)
    pooled = jnp.dot(Z, wq_ref[...], preferred_element_type=F32) + bp_ref[...]
    feats = jnp.dot(pooled, wo_ref[...], preferred_element_type=F32) + bo_ref[...]
    cnt = jnp.sum(cnt_ref[0], axis=0)
    present = cnt > 0.5
    pr = present.astype(F32)[None, :]
    r = lax.broadcasted_iota(I32, (64, 64), 0)
    c = lax.broadcasted_iota(I32, (64, 64), 1)
    pos = jnp.sum(jnp.where(c <= r, pr, 0.0), axis=1)
    perm = jnp.where((pos[None, :] - 1.0 == r.astype(F32)) & present[None, :],
                     1.0, 0.0)
    outb = jnp.dot(perm, feats, preferred_element_type=F32)
    out_ref[0] = outb[:49]


def _run_k5(zw_part, cnt_part, Wq, bp, Wo, bo, B, TPI):
    return pl.pallas_call(
        _k5_body,
        grid=(B,),
        in_specs=[
            pl.BlockSpec((1, TPI, 64, 768), lambda b: (b, 0, 0, 0)),
            pl.BlockSpec((1, TPI, 64), lambda b: (b, 0, 0)),
            pl.BlockSpec((768, 96), lambda b: (0, 0)),
            pl.BlockSpec((1, 96), lambda b: (0, 0)),
            pl.BlockSpec((96, 768), lambda b: (0, 0)),
            pl.BlockSpec((1, 768), lambda b: (0, 0)),
        ],
        out_specs=pl.BlockSpec((1, 49, 768), lambda b: (b, 0, 0)),
        out_shape=jax.ShapeDtypeStruct((B, 49, 768), F32),
    )(zw_part, cnt_part, Wq, bp, Wo, bo)


def kernel(img, W_spix, b_spix, W_patch, b_patch, w_attn, W_out, b_out):
    B, C, H, Wd = img.shape
    S = W_spix.shape[0]
    stem = W_patch.shape[0]
    patch = W_patch.shape[2]
    HP = H // patch
    HW = H * Wd
    TPI = 32 // B
    NPIX = HW // TPI

    y = np.arange(H)[:, None]
    x = np.arange(Wd)[None, :]
    pid = jnp.asarray(((y // patch) * HP + (x // patch))
                      .astype(np.int32).reshape(-1))
    off = jnp.asarray(((y % patch) * patch + (x % patch))
                      .astype(np.int32).reshape(-1))

    imgp = jnp.pad(img, ((0, 0), (0, 0), (1, 1), (1, 1)))
    cols = [imgp[:, :, dy:dy + H, dx:dx + Wd]
            for dy in range(3) for dx in range(3)]
    X = jnp.stack(cols, axis=1).reshape(B, 9 * C, HW).transpose(0, 2, 1)
    X = jnp.pad(X, ((0, 0), (0, 0), (0, 32 - 9 * C)))

    W27 = W_spix.transpose(2, 3, 1, 0).reshape(9 * C, S)
    W27 = jnp.pad(W27, ((0, 32 - 9 * C), (0, 64 - S)))
    b64 = jnp.pad(b_spix, (0, 64 - S),
                  constant_values=-1e30)[None, :]
    Wp2 = W_patch.reshape(stem, C * patch * patch)

    M = HW // 8
    labels, wbar = _run_k1(X, W27, b64, w_attn[None, :], Wp2, B, HW, M)
    labels = labels.reshape(B * HW)
    wbar = wbar.reshape(768)

    img_flat = img.reshape(B * C * HW)
    zeros256 = jnp.zeros((64, 256), F32)
    zeros768 = jnp.zeros((64, 768), F32)

    scores_part, cnt_part = _run_k2(labels, pid, off, img_flat, wbar,
                                    zeros256, B, HW, TPI, NPIX)
    attn = _run_k3(scores_part, B, TPI)
    zw_part = _run_k4(labels, pid, off, img_flat, attn, zeros768,
                      B, HW, TPI, NPIX)
    out = _run_k5(zw_part, cnt_part.reshape(B, TPI, 64), Wp2.T,
                  b_patch[None, :], W_out, b_out[None, :], B, TPI)
    return out


# speedup vs baseline: 98.0183x; 1.7136x over previous
import dataclasses
import functools

import jax
import jax.numpy as jnp
import numpy as np
from jax import lax
from jax.experimental import pallas as pl
from jax.experimental.pallas import tpu as pltpu
from jax.experimental.pallas import tpu_sc as plsc

F32 = jnp.float32
I32 = jnp.int32


def _sc_compiler_params():
    cp = pltpu.CompilerParams()
    if "needs_layout_passes" in pltpu.CompilerParams.__dataclass_fields__:
        cp = dataclasses.replace(cp, needs_layout_passes=False)
    return cp


def _k1_body(x_ref, w_ref, b_ref, wa_ref, wp_ref, lab_ref, wbar_ref):
    x = x_ref[0]
    logits = jnp.dot(w_ref[...], x, preferred_element_type=F32) + b_ref[...]
    lab_ref[0, 0, :] = jnp.argmax(logits, axis=0).astype(I32)

    @pl.when(jnp.logical_and(pl.program_id(0) == 0, pl.program_id(1) == 0))
    def _():
        wbar_ref[...] = jnp.dot(wa_ref[...], wp_ref[...],
                                preferred_element_type=F32)


def _run_k1(X, W27, b64, wa, Wp2, B, HW, M):
    nb = HW // M
    return pl.pallas_call(
        _k1_body,
        grid=(B, nb),
        in_specs=[
            pl.BlockSpec((1, 27, M), lambda b, i: (b, 0, i)),
            pl.BlockSpec((64, 27), lambda b, i: (0, 0)),
            pl.BlockSpec((64, 1), lambda b, i: (0, 0)),
            pl.BlockSpec((1, 96), lambda b, i: (0, 0)),
            pl.BlockSpec((96, 768), lambda b, i: (0, 0)),
        ],
        out_specs=[
            pl.BlockSpec((1, 1, M), lambda b, i: (b * nb + i, 0, 0)),
            pl.BlockSpec((1, 768), lambda b, i: (0, 0)),
        ],
        out_shape=[
            jax.ShapeDtypeStruct((B * nb, 1, M), I32),
            jax.ShapeDtypeStruct((1, 768), F32),
        ],
    )(X, W27, b64, wa, Wp2)


def _run_k2(labels, pid, off, img_flat, wbar, zeros256, B, HW, TPI, NPIX):
    mesh = plsc.VectorSubcoreMesh(core_axis_name="c", subcore_axis_name="s")

    @functools.partial(
        pl.kernel,
        mesh=mesh,
        compiler_params=_sc_compiler_params(),
        out_type=[
            jax.ShapeDtypeStruct((B, TPI, 64, 256), F32),
            jax.ShapeDtypeStruct((B * TPI * 64,), F32),
        ],
        scratch_types=[
            pltpu.VMEM((NPIX,), I32),
            pltpu.VMEM((NPIX,), I32),
            pltpu.VMEM((NPIX,), I32),
            pltpu.VMEM((NPIX,), F32),
            pltpu.VMEM((NPIX,), F32),
            pltpu.VMEM((NPIX,), F32),
            pltpu.VMEM((768,), F32),
            pltpu.VMEM((64, 256), F32),
            pltpu.VMEM((64,), F32),
        ],
    )
    def k2(lab_hbm, pid_hbm, off_hbm, img_hbm, wbar_hbm, z_hbm,
           s_out, c_out,
           lbl_v, pid_v, off_v, i0_v, i1_v, i2_v, wbar_v, sacc, cacc):
        wid = lax.axis_index("c") * 16 + lax.axis_index("s")
        b = wid // TPI
        part = wid % TPI
        base = part * NPIX
        pltpu.sync_copy(lab_hbm.at[pl.ds(b * HW + base, NPIX)], lbl_v)
        pltpu.sync_copy(pid_hbm.at[pl.ds(base, NPIX)], pid_v)
        pltpu.sync_copy(off_hbm.at[pl.ds(base, NPIX)], off_v)
        pltpu.sync_copy(img_hbm.at[pl.ds((b * 3 + 0) * HW + base, NPIX)], i0_v)
        pltpu.sync_copy(img_hbm.at[pl.ds((b * 3 + 1) * HW + base, NPIX)], i1_v)
        pltpu.sync_copy(img_hbm.at[pl.ds((b * 3 + 2) * HW + base, NPIX)], i2_v)
        pltpu.sync_copy(wbar_hbm, wbar_v)
        pltpu.sync_copy(z_hbm, sacc)

        @pl.loop(0, 64, step=16)
        def _(j):
            cacc[pl.ds(j, 16)] = jnp.zeros((16,), F32)

        ones = jnp.full((16,), 1.0, F32)

        @pl.loop(0, NPIX, step=16)
        def _(i):
            sl = pl.ds(i, 16)
            lbl = lbl_v[sl]
            pidv = pid_v[sl]
            offv = off_v[sl]
            w0 = plsc.load_gather(wbar_v, [offv])
            w1 = plsc.load_gather(wbar_v, [offv + 256])
            w2 = plsc.load_gather(wbar_v, [offv + 512])
            u = i0_v[sl] * w0 + i1_v[sl] * w1 + i2_v[sl] * w2
            plsc.addupdate_scatter(sacc, [lbl, pidv], u)
            plsc.addupdate_scatter(cacc, [lbl], ones)

        pltpu.sync_copy(sacc, s_out.at[b, part])
        pltpu.sync_copy(cacc, c_out.at[pl.ds(wid * 64, 64)])

    return k2(labels, pid, off, img_flat, wbar, zeros256)


def _k3_body(sp_ref, attn_ref):
    s = jnp.sum(sp_ref[0], axis=0)
    lane = lax.broadcasted_iota(I32, (64, 256), 1)
    valid = lane < 196
    neg = jnp.where(valid, s, -1e30)
    m = jnp.max(neg, axis=1, keepdims=True)
    e = jnp.where(valid, jnp.exp(s - m), 0.0)
    attn_ref[0] = e / jnp.sum(e, axis=1, keepdims=True)


def _run_k3(scores_part, B, TPI):
    return pl.pallas_call(
        _k3_body,
        grid=(B,),
        in_specs=[pl.BlockSpec((1, TPI, 64, 256), lambda b: (b, 0, 0, 0))],
        out_specs=pl.BlockSpec((1, 64, 256), lambda b: (b, 0, 0)),
        out_shape=jax.ShapeDtypeStruct((B, 64, 256), F32),
    )(scores_part)


def _run_k4(labels, pid, off, img_flat, attn, zeros768, B, HW, TPI, NPIX):
    mesh = plsc.VectorSubcoreMesh(core_axis_name="c", subcore_axis_name="s")

    @functools.partial(
        pl.kernel,
        mesh=mesh,
        compiler_params=_sc_compiler_params(),
        out_type=jax.ShapeDtypeStruct((B, TPI, 64, 768), F32),
        scratch_types=[
            pltpu.VMEM((NPIX,), I32),
            pltpu.VMEM((NPIX,), I32),
            pltpu.VMEM((NPIX,), I32),
            pltpu.VMEM((NPIX,), F32),
            pltpu.VMEM((NPIX,), F32),
            pltpu.VMEM((NPIX,), F32),
            pltpu.VMEM((64, 256), F32),
            pltpu.VMEM((64, 768), F32),
        ],
    )
    def k4(lab_hbm, pid_hbm, off_hbm, img_hbm, attn_hbm, z_hbm, zw_out,
           lbl_v, pid_v, off_v, i0_v, i1_v, i2_v, attn_v, zacc):
        wid = lax.axis_index("c") * 16 + lax.axis_index("s")
        b = wid // TPI
        part = wid % TPI
        base = part * NPIX
        pltpu.sync_copy(lab_hbm.at[pl.ds(b * HW + base, NPIX)], lbl_v)
        pltpu.sync_copy(pid_hbm.at[pl.ds(base, NPIX)], pid_v)
        pltpu.sync_copy(off_hbm.at[pl.ds(base, NPIX)], off_v)
        pltpu.sync_copy(img_hbm.at[pl.ds((b * 3 + 0) * HW + base, NPIX)], i0_v)
        pltpu.sync_copy(img_hbm.at[pl.ds((b * 3 + 1) * HW + base, NPIX)], i1_v)
        pltpu.sync_copy(img_hbm.at[pl.ds((b * 3 + 2) * HW + base, NPIX)], i2_v)
        pltpu.sync_copy(attn_hbm.at[b], attn_v)
        pltpu.sync_copy(z_hbm, zacc)

        @pl.loop(0, NPIX, step=16)
        def _(i):
            sl = pl.ds(i, 16)
            lbl = lbl_v[sl]
            pidv = pid_v[sl]
            offv = off_v[sl]
            a = plsc.load_gather(attn_v, [lbl, pidv])
            plsc.addupdate_scatter(zacc, [lbl, offv], i0_v[sl] * a)
            plsc.addupdate_scatter(zacc, [lbl, offv + 256], i1_v[sl] * a)
            plsc.addupdate_scatter(zacc, [lbl, offv + 512], i2_v[sl] * a)

        pltpu.sync_copy(zacc, zw_out.at[b, part])

    return k4(labels, pid, off, img_flat, attn, zeros768)


def _k5_body(zw_ref, cnt_ref, wq_ref, bp_ref, wo_ref, bo_ref, out_ref):
    Z = jnp.sum(zw_ref[0], axis=0)
    pooled = jnp.dot(Z, wq_ref[...], preferred_element_type=F32) + bp_ref[...]
    feats = jnp.dot(pooled, wo_ref[...], preferred_element_type=F32) + bo_ref[...]
    cnt = jnp.sum(cnt_ref[0], axis=0)
    present = cnt > 0.5
    pr = present.astype(F32)[None, :]
    r = lax.broadcasted_iota(I32, (64, 64), 0)
    c = lax.broadcasted_iota(I32, (64, 64), 1)
    pos = jnp.sum(jnp.where(c <= r, pr, 0.0), axis=1)
    perm = jnp.where((pos[None, :] - 1.0 == r.astype(F32)) & present[None, :],
                     1.0, 0.0)
    outb = jnp.dot(perm, feats, preferred_element_type=F32)
    out_ref[0] = outb[:49]


def _run_k5(zw_part, cnt_part, Wq, bp, Wo, bo, B, TPI):
    return pl.pallas_call(
        _k5_body,
        grid=(B,),
        in_specs=[
            pl.BlockSpec((1, TPI, 64, 768), lambda b: (b, 0, 0, 0)),
            pl.BlockSpec((1, TPI, 64), lambda b: (b, 0, 0)),
            pl.BlockSpec((768, 96), lambda b: (0, 0)),
            pl.BlockSpec((1, 96), lambda b: (0, 0)),
            pl.BlockSpec((96, 768), lambda b: (0, 0)),
            pl.BlockSpec((1, 768), lambda b: (0, 0)),
        ],
        out_specs=pl.BlockSpec((1, 49, 768), lambda b: (b, 0, 0)),
        out_shape=jax.ShapeDtypeStruct((B, 49, 768), F32),
    )(zw_part, cnt_part, Wq, bp, Wo, bo)


def kernel(img, W_spix, b_spix, W_patch, b_patch, w_attn, W_out, b_out):
    B, C, H, Wd = img.shape
    S = W_spix.shape[0]
    stem = W_patch.shape[0]
    patch = W_patch.shape[2]
    HP = H // patch
    HW = H * Wd
    TPI = 32 // B
    NPIX = HW // TPI

    y = np.arange(H)[:, None]
    x = np.arange(Wd)[None, :]
    pid = jnp.asarray(((y // patch) * HP + (x // patch))
                      .astype(np.int32).reshape(-1))
    off = jnp.asarray(((y % patch) * patch + (x % patch))
                      .astype(np.int32).reshape(-1))

    imgp = jnp.pad(img, ((0, 0), (0, 0), (1, 1), (1, 1)))
    cols = [imgp[:, :, dy:dy + H, dx:dx + Wd]
            for dy in range(3) for dx in range(3)]
    X = jnp.stack(cols, axis=1).reshape(B, 9 * C, HW)

    W27 = W_spix.transpose(0, 2, 3, 1).reshape(S, 9 * C)
    W27 = jnp.pad(W27, ((0, 64 - S), (0, 0)))
    b64 = jnp.pad(b_spix, (0, 64 - S),
                  constant_values=-1e30)[:, None]
    Wp2 = W_patch.reshape(stem, C * patch * patch)

    M = HW // 8
    labels, wbar = _run_k1(X, W27, b64, w_attn[None, :], Wp2, B, HW, M)
    labels = labels.reshape(B * HW)
    wbar = wbar.reshape(768)

    img_flat = img.reshape(B * C * HW)
    zeros256 = jnp.zeros((64, 256), F32)
    zeros768 = jnp.zeros((64, 768), F32)

    scores_part, cnt_part = _run_k2(labels, pid, off, img_flat, wbar,
                                    zeros256, B, HW, TPI, NPIX)
    attn = _run_k3(scores_part, B, TPI)
    zw_part = _run_k4(labels, pid, off, img_flat, attn, zeros768,
                      B, HW, TPI, NPIX)
    out = _run_k5(zw_part, cnt_part.reshape(B, TPI, 64), Wp2.T,
                  b_patch[None, :], W_out, b_out[None, :], B, TPI)
    return out
